# baseline (speedup 1.0000x reference)
import jax
import jax.numpy as jnp
from jax import lax
from jax.experimental import pallas as pl

_INTERPRET = False

_NE = 512


def _bf(a):
    return a.astype(jnp.bfloat16)


def _dot(a, b):
    return lax.dot_general(_bf(a), _bf(b), (((1,), (0,)), ((), ())),
                           preferred_element_type=jnp.float32)


def _dotf(a, b):
    return lax.dot_general(a, b, (((1,), (0,)), ((), ())),
                           preferred_element_type=jnp.float32)


def _cat(parts):
    return jnp.concatenate(parts, axis=1)


def _mk_conv3_body(pre_relu, post_relu):
    def body(x_ref, w_ref, b_ref, o_ref):
        x = x_ref[0]
        if pre_relu:
            x = jnp.maximum(x, 0.0)
        t = o_ref.shape[1]
        xcol = _cat([x[0:t], x[1:1 + t], x[2:2 + t]])
        acc = _dot(xcol, w_ref[...]) + b_ref[0][None, :]
        if post_relu:
            acc = jnp.maximum(acc, 0.0)
        o_ref[0] = acc
    return body


def _conv3(x, wcol, b, pre_relu=False, post_relu=False):
    B, t_pad, cin = x.shape
    t = t_pad - 2
    cout = wcol.shape[1]
    return pl.pallas_call(
        _mk_conv3_body(pre_relu, post_relu),
        grid=(B,),
        in_specs=[
            pl.BlockSpec((1, t_pad, cin), lambda i: (i, 0, 0)),
            pl.BlockSpec((3 * cin, cout), lambda i: (0, 0)),
            pl.BlockSpec((1, cout), lambda i: (0, 0)),
        ],
        out_specs=pl.BlockSpec((1, t, cout), lambda i: (i, 0, 0)),
        out_shape=jax.ShapeDtypeStruct((B, t, cout), jnp.float32),
        interpret=_INTERPRET,
    )(x, wcol, b.reshape(1, -1))


def _mk_sconv_body(cin, pre_relu, post_relu):
    def body(p_ref, w_ref, b_ref, o_ref):
        pp = p_ref[0]
        if pre_relu:
            pp = jnp.maximum(pp, 0.0)
        t = o_ref.shape[1]
        c = cin
        xcol = _cat([pp[0:t, c:2 * c], pp[1:1 + t, 0:c],
                     pp[1:1 + t, c:2 * c], pp[2:2 + t, 0:c]])
        acc = _dot(xcol, w_ref[...]) + b_ref[0][None, :]
        if post_relu:
            acc = jnp.maximum(acc, 0.0)
        o_ref[0] = acc
    return body


def _sconv(pf, wcol, b, pre_relu=False, post_relu=False):
    B, t_pad, c2 = pf.shape
    t = t_pad - 2
    cin = c2 // 2
    cout = wcol.shape[1]
    return pl.pallas_call(
        _mk_sconv_body(cin, pre_relu, post_relu),
        grid=(B,),
        in_specs=[
            pl.BlockSpec((1, t_pad, c2), lambda i: (i, 0, 0)),
            pl.BlockSpec((4 * cin, cout), lambda i: (0, 0)),
            pl.BlockSpec((1, cout), lambda i: (0, 0)),
        ],
        out_specs=pl.BlockSpec((1, t, cout), lambda i: (i, 0, 0)),
        out_shape=jax.ShapeDtypeStruct((B, t, cout), jnp.float32),
        interpret=_INTERPRET,
    )(pf, wcol, b.reshape(1, -1))


def _mk_tconv_body(pre_relu, post_relu):
    def body(x_ref, w_ref, b_ref, o_ref):
        x = x_ref[0]
        if pre_relu:
            x = jnp.maximum(x, 0.0)
        t = o_ref.shape[1]
        c = x.shape[1]
        z = jnp.zeros((t, c), jnp.float32)
        evcol = _cat([x[0:t], z, x[1:1 + t], z])
        odcol = _cat([z, x[1:1 + t], z, x[2:2 + t]])
        bb = b_ref[0][None, :]
        ev = _dot(evcol, w_ref[...]) + bb
        od = _dot(odcol, w_ref[...]) + bb
        acc = _cat([ev, od])
        if post_relu:
            acc = jnp.maximum(acc, 0.0)
        o_ref[0] = acc
    return body


def _tconv(x, wcol, b, pre_relu=False, post_relu=False):
    B, t_pad, cin = x.shape
    t = t_pad - 2
    cout = wcol.shape[1]
    return pl.pallas_call(
        _mk_tconv_body(pre_relu, post_relu),
        grid=(B,),
        in_specs=[
            pl.BlockSpec((1, t_pad, cin), lambda i: (i, 0, 0)),
            pl.BlockSpec((4 * cin, cout), lambda i: (0, 0)),
            pl.BlockSpec((1, cout), lambda i: (0, 0)),
        ],
        out_specs=pl.BlockSpec((1, t, 2 * cout), lambda i: (i, 0, 0)),
        out_shape=jax.ShapeDtypeStruct((B, t, 2 * cout), jnp.float32),
        interpret=_INTERPRET,
    )(x, wcol, b.reshape(1, -1))


def _resblock_body(x_ref, w1_ref, b1_ref, w2_ref, b2_ref, o_ref):
    x = x_ref[0]
    t = o_ref.shape[1]
    xa = jnp.maximum(x, 0.0)
    xcol = _cat([xa[0:t], xa[1:1 + t], xa[2:2 + t]])
    a = jnp.maximum(_dot(xcol, w1_ref[...]) + b1_ref[0][None, :], 0.0)
    o_ref[0] = x[1:1 + t] + (_dot(a, w2_ref[...]) + b2_ref[0][None, :])


def _resblock(x, wcol1, b1, w2, b2):
    B, t, c = x.shape
    m = wcol1.shape[1]
    xp = jnp.pad(x, ((0, 0), (1, 1), (0, 0)))
    return pl.pallas_call(
        _resblock_body,
        grid=(B,),
        in_specs=[
            pl.BlockSpec((1, t + 2, c), lambda i: (i, 0, 0)),
            pl.BlockSpec((3 * c, m), lambda i: (0, 0)),
            pl.BlockSpec((1, m), lambda i: (0, 0)),
            pl.BlockSpec((m, c), lambda i: (0, 0)),
            pl.BlockSpec((1, c), lambda i: (0, 0)),
        ],
        out_specs=pl.BlockSpec((1, t, c), lambda i: (i, 0, 0)),
        out_shape=jax.ShapeDtypeStruct((B, t, c), jnp.float32),
        interpret=_INTERPRET,
    )(xp, wcol1, b1.reshape(1, -1), w2, b2.reshape(1, -1))


def _quant_common(z, emb_ref, embt_ref, en_ref, q_ref, diff_ref):
    d = en_ref[0][None, :] - 2.0 * _dot(z, emb_ref[...])
    m = jnp.min(d, axis=1, keepdims=True)
    iota = lax.broadcasted_iota(jnp.int32, d.shape, 1)
    idx = jnp.min(jnp.where(d == m, iota, _NE), axis=1, keepdims=True)
    oh = (iota == idx).astype(jnp.float32)
    q = _dotf(oh, embt_ref[...])
    q_ref[0] = q

    @pl.when(pl.program_id(0) == 0)
    def _():
        diff_ref[...] = jnp.zeros((1, 1), jnp.float32)

    diff_ref[...] += jnp.sum((q - z) ** 2).reshape(1, 1)


def _qtop_body(x_ref, wq_ref, bq_ref, emb_ref, embt_ref, en_ref, q_ref,
               diff_ref):
    z = _dot(jnp.maximum(x_ref[0], 0.0), wq_ref[...]) + bq_ref[0][None, :]
    _quant_common(z, emb_ref, embt_ref, en_ref, q_ref, diff_ref)


def _quant_top(x, wq, bq, embed):
    B, t, c = x.shape
    ed = wq.shape[1]
    en = jnp.sum(embed * embed, axis=0).reshape(1, -1)
    q, diff = pl.pallas_call(
        _qtop_body,
        grid=(B,),
        in_specs=[
            pl.BlockSpec((1, t, c), lambda i: (i, 0, 0)),
            pl.BlockSpec((c, ed), lambda i: (0, 0)),
            pl.BlockSpec((1, ed), lambda i: (0, 0)),
            pl.BlockSpec((ed, _NE), lambda i: (0, 0)),
            pl.BlockSpec((_NE, ed), lambda i: (0, 0)),
            pl.BlockSpec((1, _NE), lambda i: (0, 0)),
        ],
        out_specs=[
            pl.BlockSpec((1, t, ed), lambda i: (i, 0, 0)),
            pl.BlockSpec((1, 1), lambda i: (0, 0)),
        ],
        out_shape=[
            jax.ShapeDtypeStruct((B, t, ed), jnp.float32),
            jax.ShapeDtypeStruct((1, 1), jnp.float32),
        ],
        interpret=_INTERPRET,
    )(x, wq, bq.reshape(1, -1), embed, embed.T, en)
    return q, diff[0, 0]


def _qbot_body(x1_ref, x2_ref, wq_ref, bq_ref, emb_ref, embt_ref,
               en_ref, q_ref, diff_ref):
    xcat = _cat([x1_ref[0], jnp.maximum(x2_ref[0], 0.0)])
    z = _dot(xcat, wq_ref[...]) + bq_ref[0][None, :]
    _quant_common(z, emb_ref, embt_ref, en_ref, q_ref, diff_ref)


def _quant_bot(x1, x2, wq, bq, embed):
    B, t, c1 = x1.shape
    c2 = x2.shape[2]
    ed = wq.shape[1]
    en = jnp.sum(embed * embed, axis=0).reshape(1, -1)
    q, diff = pl.pallas_call(
        _qbot_body,
        grid=(B,),
        in_specs=[
            pl.BlockSpec((1, t, c1), lambda i: (i, 0, 0)),
            pl.BlockSpec((1, t, c2), lambda i: (i, 0, 0)),
            pl.BlockSpec((c1 + c2, ed), lambda i: (0, 0)),
            pl.BlockSpec((1, ed), lambda i: (0, 0)),
            pl.BlockSpec((ed, _NE), lambda i: (0, 0)),
            pl.BlockSpec((_NE, ed), lambda i: (0, 0)),
            pl.BlockSpec((1, _NE), lambda i: (0, 0)),
        ],
        out_specs=[
            pl.BlockSpec((1, t, ed), lambda i: (i, 0, 0)),
            pl.BlockSpec((1, 1), lambda i: (0, 0)),
        ],
        out_shape=[
            jax.ShapeDtypeStruct((B, t, ed), jnp.float32),
            jax.ShapeDtypeStruct((1, 1), jnp.float32),
        ],
        interpret=_INTERPRET,
    )(x1, x2, wq, bq.reshape(1, -1), embed, embed.T, en)
    return q, diff[0, 0]


def _pad1(x):
    return jnp.pad(x, ((0, 0), (1, 1), (0, 0)))


def _fold(x):
    b, t, c = x.shape
    return x.reshape(b, t // 2, 2 * c)


def _wcol_conv(w):
    k = w.shape[2]
    wt = jnp.transpose(w, (2, 1, 0))
    return jnp.concatenate([wt[j] for j in range(k)], axis=0)


def _wcol_tconv(w):
    wf = jnp.flip(w, -1)
    return jnp.concatenate([wf[:, :, j] for j in range(4)], axis=0)


def kernel(input, params):
    p = params
    B = input.shape[0]
    x = jnp.transpose(input, (0, 2, 1))

    h = _sconv(_pad1(_fold(x)), _wcol_conv(p['eb_w0']), p['eb_b0'],
               post_relu=True)
    h = _sconv(_pad1(_fold(h)), _wcol_conv(p['eb_w1']), p['eb_b1'],
               post_relu=True)
    h = _conv3(_pad1(h), _wcol_conv(p['eb_w2']), p['eb_b2'])
    for i in range(2):
        h = _resblock(h, _wcol_conv(p['eb_r%d_w1' % i]), p['eb_r%d_b1' % i],
                      p['eb_r%d_w2' % i][:, :, 0].T, p['eb_r%d_b2' % i])
    enc_b_pre = h

    h = _sconv(_pad1(_fold(enc_b_pre)), _wcol_conv(p['et_w0']), p['et_b0'],
               pre_relu=True, post_relu=True)
    h = _conv3(_pad1(h), _wcol_conv(p['et_w1']), p['et_b1'])
    for i in range(2):
        h = _resblock(h, _wcol_conv(p['et_r%d_w1' % i]), p['et_r%d_b1' % i],
                      p['et_r%d_w2' % i][:, :, 0].T, p['et_r%d_b2' % i])

    quant_t, dt_sum = _quant_top(h, p['qct_w'][:, :, 0].T, p['qct_b'],
                                 p['embed_t'])

    h = _conv3(_pad1(quant_t), _wcol_conv(p['dt_w0']), p['dt_b0'])
    for i in range(2):
        h = _resblock(h, _wcol_conv(p['dt_r%d_w1' % i]), p['dt_r%d_b1' % i],
                      p['dt_r%d_w2' % i][:, :, 0].T, p['dt_r%d_b2' % i])
    h = _tconv(_pad1(h), _wcol_tconv(p['dt_wt']), p['dt_bt'],
               pre_relu=True)
    dec_t = h.reshape(B, 2048, 64)

    quant_b, db_sum = _quant_bot(dec_t, enc_b_pre, p['qcb_w'][:, :, 0].T,
                                 p['qcb_b'], p['embed_b'])

    up = _tconv(_pad1(quant_t), _wcol_tconv(p['up_wt']), p['up_bt'])
    up_t = up.reshape(B, 2048, 64)
    cat = jnp.concatenate([up_t, quant_b], axis=2)
    h = _conv3(_pad1(cat), _wcol_conv(p['d_w0']), p['d_b0'])
    for i in range(2):
        h = _resblock(h, _wcol_conv(p['d_r%d_w1' % i]), p['d_r%d_b1' % i],
                      p['d_r%d_w2' % i][:, :, 0].T, p['d_r%d_b2' % i])
    h = _tconv(_pad1(h), _wcol_tconv(p['d_wt1']), p['d_bt1'],
               pre_relu=True, post_relu=True)
    h = h.reshape(B, 4096, 64)
    h = _tconv(_pad1(h), _wcol_tconv(p['d_wt2']), p['d_bt2'])
    dec = h.reshape(B, 8192, 3).transpose(0, 2, 1)

    diff = (dt_sum / (B * 1024 * 64) + db_sum / (B * 2048 * 64)).reshape(1)
    return dec, diff


# speedup vs baseline: 1.3868x; 1.3868x over previous
import jax
import jax.numpy as jnp
from jax import lax
from jax.experimental import pallas as pl

_INTERPRET = False

_NE = 512


def _bf(a):
    return a.astype(jnp.bfloat16)


def _dot(a, b):
    return lax.dot_general(_bf(a), _bf(b), (((1,), (0,)), ((), ())),
                           preferred_element_type=jnp.float32)


def _dotf(a, b):
    return lax.dot_general(a, b, (((1,), (0,)), ((), ())),
                           preferred_element_type=jnp.float32)


def _cat(parts):
    return jnp.concatenate(parts, axis=1)


def _cat3(x, t):
    return _cat([x[0:t], x[1:1 + t], x[2:2 + t]])


def _scol(pp, t, c):
    return _cat([pp[0:t, c:2 * c], pp[1:1 + t, 0:c],
                 pp[1:1 + t, c:2 * c], pp[2:2 + t, 0:c]])


def _vpad(h):
    z = jnp.zeros((1, h.shape[1]), jnp.float32)
    return jnp.concatenate([z, h, z], axis=0)


def _res_step(h, t, w1col_ref, b1_ref, w2_ref, b2_ref):
    hp = _vpad(h)
    ha = jnp.maximum(hp, 0.0)
    a = jnp.maximum(_dot(_cat3(ha, t), w1col_ref[...]) + b1_ref[0][None, :],
                    0.0)
    return h + (_dot(a, w2_ref[...]) + b2_ref[0][None, :])


def _quant_common(z, emb_ref, embt_ref, en_ref, q_ref, diff_ref):
    d = en_ref[0][None, :] - 2.0 * _dot(z, emb_ref[...])
    m = jnp.min(d, axis=1, keepdims=True)
    iota = lax.broadcasted_iota(jnp.int32, d.shape, 1)
    idx = jnp.min(jnp.where(d == m, iota, _NE), axis=1, keepdims=True)
    oh = (iota == idx).astype(jnp.float32)
    q = _dotf(oh, embt_ref[...])
    q_ref[0] = q

    @pl.when(pl.program_id(0) == 0)
    def _():
        diff_ref[...] = jnp.zeros((1, 1), jnp.float32)

    diff_ref[...] += jnp.sum((q - z) ** 2).reshape(1, 1)


def _wspec(*shape):
    n = len(shape)
    return pl.BlockSpec(shape, lambda i, _n=n: (0,) * _n)


def _bspec(t, c):
    return pl.BlockSpec((1, t, c), lambda i: (i, 0, 0))


def _eb0_body(p_ref, w_ref, b_ref, o_ref):
    t = o_ref.shape[1]
    xcol = _scol(p_ref[0], t, 3)
    o_ref[0] = jnp.maximum(_dot(xcol, w_ref[...]) + b_ref[0][None, :], 0.0)


def _encb_body(p_ref, w0_ref, b0_ref, w1_ref, b1_ref,
               r0w1_ref, r0b1_ref, r0w2_ref, r0b2_ref,
               r1w1_ref, r1b1_ref, r1w2_ref, r1b2_ref, o_ref):
    t = o_ref.shape[1]
    xcol = _scol(p_ref[0], t, 64)
    h = jnp.maximum(_dot(xcol, w0_ref[...]) + b0_ref[0][None, :], 0.0)
    h = _dot(_cat3(_vpad(h), t), w1_ref[...]) + b1_ref[0][None, :]
    h = _res_step(h, t, r0w1_ref, r0b1_ref, r0w2_ref, r0b2_ref)
    h = _res_step(h, t, r1w1_ref, r1b1_ref, r1w2_ref, r1b2_ref)
    o_ref[0] = h


def _enct_body(p_ref, w0_ref, b0_ref, w1_ref, b1_ref,
               r0w1_ref, r0b1_ref, r0w2_ref, r0b2_ref,
               r1w1_ref, r1b1_ref, r1w2_ref, r1b2_ref,
               wq_ref, bq_ref, emb_ref, embt_ref, en_ref, q_ref, diff_ref):
    t = q_ref.shape[1]
    pp = jnp.maximum(p_ref[0], 0.0)
    xcol = _scol(pp, t, 128)
    h = jnp.maximum(_dot(xcol, w0_ref[...]) + b0_ref[0][None, :], 0.0)
    h = _dot(_cat3(_vpad(h), t), w1_ref[...]) + b1_ref[0][None, :]
    h = _res_step(h, t, r0w1_ref, r0b1_ref, r0w2_ref, r0b2_ref)
    h = _res_step(h, t, r1w1_ref, r1b1_ref, r1w2_ref, r1b2_ref)
    z = _dot(jnp.maximum(h, 0.0), wq_ref[...]) + bq_ref[0][None, :]
    _quant_common(z, emb_ref, embt_ref, en_ref, q_ref, diff_ref)


def _dect_body(x_ref, w0_ref, b0_ref,
               r0w1_ref, r0b1_ref, r0w2_ref, r0b2_ref,
               r1w1_ref, r1b1_ref, r1w2_ref, r1b2_ref,
               wt_ref, bt_ref, o_ref):
    t = o_ref.shape[1]
    h = _dot(_cat3(x_ref[0], t), w0_ref[...]) + b0_ref[0][None, :]
    h = _res_step(h, t, r0w1_ref, r0b1_ref, r0w2_ref, r0b2_ref)
    h = _res_step(h, t, r1w1_ref, r1b1_ref, r1w2_ref, r1b2_ref)
    hp = _vpad(jnp.maximum(h, 0.0))
    c = h.shape[1]
    z = jnp.zeros((t, c), jnp.float32)
    bb = bt_ref[0][None, :]
    ev = _dot(_cat([hp[0:t], z, hp[1:1 + t], z]), wt_ref[...]) + bb
    od = _dot(_cat([z, hp[1:1 + t], z, hp[2:2 + t]]), wt_ref[...]) + bb
    o_ref[0] = _cat([ev, od])


def _qbot_body(x1_ref, x2_ref, wq_ref, bq_ref, emb_ref, embt_ref,
               en_ref, q_ref, diff_ref):
    xcat = _cat([x1_ref[0], jnp.maximum(x2_ref[0], 0.0)])
    z = _dot(xcat, wq_ref[...]) + bq_ref[0][None, :]
    _quant_common(z, emb_ref, embt_ref, en_ref, q_ref, diff_ref)


def _mk_tconv2_body(post_relu):
    def body(x_ref, wev_ref, wod_ref, b_ref, o_ref):
        x = x_ref[0]
        t = o_ref.shape[1]
        bb = b_ref[0][None, :]
        ev = _dot(_cat([x[1:1 + t], x[0:t]]), wev_ref[...]) + bb
        od = _dot(_cat([x[1:1 + t], x[2:2 + t]]), wod_ref[...]) + bb
        acc = _cat([ev, od])
        if post_relu:
            acc = jnp.maximum(acc, 0.0)
        o_ref[0] = acc
    return body


def _dec_body(x_ref, w0_ref, b0_ref,
              r0w1_ref, r0b1_ref, r0w2_ref, r0b2_ref,
              r1w1_ref, r1b1_ref, r1w2_ref, r1b2_ref,
              wev_ref, wod_ref, bt_ref, o_ref):
    t = o_ref.shape[1]
    h = _dot(_cat3(x_ref[0], t), w0_ref[...]) + b0_ref[0][None, :]
    h = _res_step(h, t, r0w1_ref, r0b1_ref, r0w2_ref, r0b2_ref)
    h = _res_step(h, t, r1w1_ref, r1b1_ref, r1w2_ref, r1b2_ref)
    hp = _vpad(jnp.maximum(h, 0.0))
    bb = bt_ref[0][None, :]
    ev = _dot(_cat([hp[1:1 + t], hp[0:t]]), wev_ref[...]) + bb
    od = _dot(_cat([hp[1:1 + t], hp[2:2 + t]]), wod_ref[...]) + bb
    o_ref[0] = jnp.maximum(_cat([ev, od]), 0.0)


def _pad1(x):
    return jnp.pad(x, ((0, 0), (1, 1), (0, 0)))


def _fold(x):
    b, t, c = x.shape
    return x.reshape(b, t // 2, 2 * c)


def _wcol_conv(w):
    k = w.shape[2]
    wt = jnp.transpose(w, (2, 1, 0))
    return jnp.concatenate([wt[j] for j in range(k)], axis=0)


def _wcol_tconv(w):
    wf = jnp.flip(w, -1)
    return jnp.concatenate([wf[:, :, j] for j in range(4)], axis=0)


def _w2tap(w):
    wev = jnp.concatenate([w[:, :, 1], w[:, :, 3]], axis=0)
    wod = jnp.concatenate([w[:, :, 2], w[:, :, 0]], axis=0)
    return wev, wod


def _b2(b):
    return b.reshape(1, -1)


def _rb(p, pre, i):
    return (_wcol_conv(p['%s_r%d_w1' % (pre, i)]), _b2(p['%s_r%d_b1' % (pre, i)]),
            p['%s_r%d_w2' % (pre, i)][:, :, 0].T, _b2(p['%s_r%d_b2' % (pre, i)]))


def kernel(input, params):
    p = params
    B = input.shape[0]
    f32 = jnp.float32
    x = jnp.transpose(input, (0, 2, 1))

    h = pl.pallas_call(
        _eb0_body, grid=(B,),
        in_specs=[_bspec(4098, 6), _wspec(12, 64), _wspec(1, 64)],
        out_specs=_bspec(4096, 64),
        out_shape=jax.ShapeDtypeStruct((B, 4096, 64), f32),
        interpret=_INTERPRET,
    )(_pad1(_fold(x)), _wcol_conv(p['eb_w0']), _b2(p['eb_b0']))

    enc_b_pre = pl.pallas_call(
        _encb_body, grid=(B,),
        in_specs=[_bspec(2050, 128), _wspec(256, 128), _wspec(1, 128),
                  _wspec(384, 128), _wspec(1, 128),
                  _wspec(384, 32), _wspec(1, 32), _wspec(32, 128), _wspec(1, 128),
                  _wspec(384, 32), _wspec(1, 32), _wspec(32, 128), _wspec(1, 128)],
        out_specs=_bspec(2048, 128),
        out_shape=jax.ShapeDtypeStruct((B, 2048, 128), f32),
        interpret=_INTERPRET,
    )(_pad1(_fold(h)), _wcol_conv(p['eb_w1']), _b2(p['eb_b1']),
      _wcol_conv(p['eb_w2']), _b2(p['eb_b2']),
      *_rb(p, 'eb', 0), *_rb(p, 'eb', 1))

    en_t = jnp.sum(p['embed_t'] * p['embed_t'], axis=0).reshape(1, -1)
    quant_t, dt_sum = pl.pallas_call(
        _enct_body, grid=(B,),
        in_specs=[_bspec(1026, 256), _wspec(512, 64), _wspec(1, 64),
                  _wspec(192, 128), _wspec(1, 128),
                  _wspec(384, 32), _wspec(1, 32), _wspec(32, 128), _wspec(1, 128),
                  _wspec(384, 32), _wspec(1, 32), _wspec(32, 128), _wspec(1, 128),
                  _wspec(128, 64), _wspec(1, 64),
                  _wspec(64, _NE), _wspec(_NE, 64), _wspec(1, _NE)],
        out_specs=[_bspec(1024, 64), pl.BlockSpec((1, 1), lambda i: (0, 0))],
        out_shape=[jax.ShapeDtypeStruct((B, 1024, 64), f32),
                   jax.ShapeDtypeStruct((1, 1), f32)],
        interpret=_INTERPRET,
    )(_pad1(_fold(enc_b_pre)), _wcol_conv(p['et_w0']), _b2(p['et_b0']),
      _wcol_conv(p['et_w1']), _b2(p['et_b1']),
      *_rb(p, 'et', 0), *_rb(p, 'et', 1),
      p['qct_w'][:, :, 0].T, _b2(p['qct_b']),
      p['embed_t'], p['embed_t'].T, en_t)

    dec_t_f = pl.pallas_call(
        _dect_body, grid=(B,),
        in_specs=[_bspec(1026, 64), _wspec(192, 128), _wspec(1, 128),
                  _wspec(384, 32), _wspec(1, 32), _wspec(32, 128), _wspec(1, 128),
                  _wspec(384, 32), _wspec(1, 32), _wspec(32, 128), _wspec(1, 128),
                  _wspec(512, 64), _wspec(1, 64)],
        out_specs=_bspec(1024, 128),
        out_shape=jax.ShapeDtypeStruct((B, 1024, 128), f32),
        interpret=_INTERPRET,
    )(_pad1(quant_t), _wcol_conv(p['dt_w0']), _b2(p['dt_b0']),
      *_rb(p, 'dt', 0), *_rb(p, 'dt', 1),
      _wcol_tconv(p['dt_wt']), _b2(p['dt_bt']))
    dec_t = dec_t_f.reshape(B, 2048, 64)

    en_b = jnp.sum(p['embed_b'] * p['embed_b'], axis=0).reshape(1, -1)
    quant_b, db_sum = pl.pallas_call(
        _qbot_body, grid=(B,),
        in_specs=[_bspec(2048, 64), _bspec(2048, 128),
                  _wspec(192, 64), _wspec(1, 64),
                  _wspec(64, _NE), _wspec(_NE, 64), _wspec(1, _NE)],
        out_specs=[_bspec(2048, 64), pl.BlockSpec((1, 1), lambda i: (0, 0))],
        out_shape=[jax.ShapeDtypeStruct((B, 2048, 64), f32),
                   jax.ShapeDtypeStruct((1, 1), f32)],
        interpret=_INTERPRET,
    )(dec_t, enc_b_pre, p['qcb_w'][:, :, 0].T, _b2(p['qcb_b']),
      p['embed_b'], p['embed_b'].T, en_b)

    upev, upod = _w2tap(p['up_wt'])
    up = pl.pallas_call(
        _mk_tconv2_body(False), grid=(B,),
        in_specs=[_bspec(1026, 64), _wspec(128, 64), _wspec(128, 64),
                  _wspec(1, 64)],
        out_specs=_bspec(1024, 128),
        out_shape=jax.ShapeDtypeStruct((B, 1024, 128), f32),
        interpret=_INTERPRET,
    )(_pad1(quant_t), upev, upod, _b2(p['up_bt']))
    up_t = up.reshape(B, 2048, 64)

    cat = jnp.concatenate([up_t, quant_b], axis=2)
    d1ev, d1od = _w2tap(p['d_wt1'])
    h = pl.pallas_call(
        _dec_body, grid=(B,),
        in_specs=[_bspec(2050, 128), _wspec(384, 128), _wspec(1, 128),
                  _wspec(384, 32), _wspec(1, 32), _wspec(32, 128), _wspec(1, 128),
                  _wspec(384, 32), _wspec(1, 32), _wspec(32, 128), _wspec(1, 128),
                  _wspec(256, 64), _wspec(256, 64), _wspec(1, 64)],
        out_specs=_bspec(2048, 128),
        out_shape=jax.ShapeDtypeStruct((B, 2048, 128), f32),
        interpret=_INTERPRET,
    )(_pad1(cat), _wcol_conv(p['d_w0']), _b2(p['d_b0']),
      *_rb(p, 'd', 0), *_rb(p, 'd', 1),
      d1ev, d1od, _b2(p['d_bt1']))
    h = h.reshape(B, 4096, 64)

    d2ev, d2od = _w2tap(p['d_wt2'])
    h = pl.pallas_call(
        _mk_tconv2_body(False), grid=(B,),
        in_specs=[_bspec(4098, 64), _wspec(128, 3), _wspec(128, 3),
                  _wspec(1, 3)],
        out_specs=_bspec(4096, 6),
        out_shape=jax.ShapeDtypeStruct((B, 4096, 6), f32),
        interpret=_INTERPRET,
    )(_pad1(h), d2ev, d2od, _b2(p['d_bt2']))
    dec = h.reshape(B, 8192, 3).transpose(0, 2, 1)

    diff = (dt_sum[0, 0] / (B * 1024 * 64)
            + db_sum[0, 0] / (B * 2048 * 64)).reshape(1)
    return dec, diff


# speedup vs baseline: 1.5512x; 1.1185x over previous
import jax
import jax.numpy as jnp
from jax import lax
from jax.experimental import pallas as pl

_INTERPRET = False

_NE = 512


def _bf(a):
    return a.astype(jnp.bfloat16)


def _dot(a, b):
    return lax.dot_general(_bf(a), _bf(b), (((1,), (0,)), ((), ())),
                           preferred_element_type=jnp.float32)


def _dotf(a, b):
    return lax.dot_general(a, b, (((1,), (0,)), ((), ())),
                           preferred_element_type=jnp.float32)


def _cat(parts):
    return jnp.concatenate(parts, axis=1)


def _cat3(x, t):
    return _cat([x[0:t], x[1:1 + t], x[2:2 + t]])


def _scol(pp, t, c):
    return _cat([pp[0:t, c:2 * c], pp[1:1 + t, 0:c],
                 pp[1:1 + t, c:2 * c], pp[2:2 + t, 0:c]])


def _vpad(h):
    z = jnp.zeros((1, h.shape[1]), jnp.float32)
    return jnp.concatenate([z, h, z], axis=0)


def _res_step(h, t, w1col_ref, b1_ref, w2_ref, b2_ref):
    hp = _vpad(h)
    ha = jnp.maximum(hp, 0.0)
    a = jnp.maximum(_dot(_cat3(ha, t), w1col_ref[...]) + b1_ref[0][None, :],
                    0.0)
    return h + (_dot(a, w2_ref[...]) + b2_ref[0][None, :])


def _quant_common(z, emb_ref, embt_ref, en_ref, q_ref, diff_ref):
    d = en_ref[0][None, :] - 2.0 * _dot(z, emb_ref[...])
    m = jnp.min(d, axis=1, keepdims=True)
    iota = lax.broadcasted_iota(jnp.int32, d.shape, 1)
    idx = jnp.min(jnp.where(d == m, iota, _NE), axis=1, keepdims=True)
    oh = (iota == idx).astype(jnp.float32)
    q = _dotf(oh, embt_ref[...])
    q_ref[0] = q

    @pl.when(pl.program_id(0) == 0)
    def _():
        diff_ref[...] = jnp.zeros((1, 1), jnp.float32)

    diff_ref[...] += jnp.sum((q - z) ** 2).reshape(1, 1)


def _wspec(*shape):
    n = len(shape)
    return pl.BlockSpec(shape, lambda i, _n=n: (0,) * _n)


def _bspec(t, c):
    return pl.BlockSpec((1, t, c), lambda i: (i, 0, 0))


def _eb0_body(p_ref, w_ref, b_ref, o_ref):
    t = o_ref.shape[1]
    xcol = _scol(p_ref[0], t, 3)
    o_ref[0] = jnp.maximum(_dot(xcol, w_ref[...]) + b_ref[0][None, :], 0.0)


def _encb_body(p_ref, w0_ref, b0_ref, w1_ref, b1_ref,
               r0w1_ref, r0b1_ref, r0w2_ref, r0b2_ref,
               r1w1_ref, r1b1_ref, r1w2_ref, r1b2_ref, o_ref):
    t = o_ref.shape[1]
    xcol = _scol(p_ref[0], t, 64)
    h = jnp.maximum(_dot(xcol, w0_ref[...]) + b0_ref[0][None, :], 0.0)
    h = _dot(_cat3(_vpad(h), t), w1_ref[...]) + b1_ref[0][None, :]
    h = _res_step(h, t, r0w1_ref, r0b1_ref, r0w2_ref, r0b2_ref)
    h = _res_step(h, t, r1w1_ref, r1b1_ref, r1w2_ref, r1b2_ref)
    o_ref[0] = h


def _enct_body(p_ref, w0_ref, b0_ref, w1_ref, b1_ref,
               r0w1_ref, r0b1_ref, r0w2_ref, r0b2_ref,
               r1w1_ref, r1b1_ref, r1w2_ref, r1b2_ref,
               wq_ref, bq_ref, emb_ref, embt_ref, en_ref, q_ref, diff_ref):
    t = q_ref.shape[1]
    pp = jnp.maximum(p_ref[0], 0.0)
    xcol = _scol(pp, t, 128)
    h = jnp.maximum(_dot(xcol, w0_ref[...]) + b0_ref[0][None, :], 0.0)
    h = _dot(_cat3(_vpad(h), t), w1_ref[...]) + b1_ref[0][None, :]
    h = _res_step(h, t, r0w1_ref, r0b1_ref, r0w2_ref, r0b2_ref)
    h = _res_step(h, t, r1w1_ref, r1b1_ref, r1w2_ref, r1b2_ref)
    z = _dot(jnp.maximum(h, 0.0), wq_ref[...]) + bq_ref[0][None, :]
    _quant_common(z, emb_ref, embt_ref, en_ref, q_ref, diff_ref)


def _mega_body(x_ref, p2_ref,
               w0_ref, b0_ref,
               r0w1_ref, r0b1_ref, r0w2_ref, r0b2_ref,
               r1w1_ref, r1b1_ref, r1w2_ref, r1b2_ref,
               wt_ref, bt_ref,
               wq_ref, bq_ref, emb_ref, embt_ref, en_ref,
               upev_ref, upod_ref, bup_ref,
               wd0_ref, bd0_ref,
               s0w1_ref, s0b1_ref, s0w2_ref, s0b2_ref,
               s1w1_ref, s1b1_ref, s1w2_ref, s1b2_ref,
               d1ev_ref, d1od_ref, bt1_ref,
               o_ref, diff_ref):
    t = o_ref.shape[1]
    x = x_ref[0]

    h = _dot(_cat3(x, t), w0_ref[...]) + b0_ref[0][None, :]
    h = _res_step(h, t, r0w1_ref, r0b1_ref, r0w2_ref, r0b2_ref)
    h = _res_step(h, t, r1w1_ref, r1b1_ref, r1w2_ref, r1b2_ref)
    hp = _vpad(jnp.maximum(h, 0.0))
    z128 = jnp.zeros((t, 128), jnp.float32)
    bb = bt_ref[0][None, :]
    dec_ev = _dot(_cat([hp[0:t], z128, hp[1:1 + t], z128]), wt_ref[...]) + bb
    dec_od = _dot(_cat([z128, hp[1:1 + t], z128, hp[2:2 + t]]), wt_ref[...]) + bb

    encbf = jnp.maximum(p2_ref[0][1:1 + t], 0.0)
    bqv = bq_ref[0][None, :]
    z_ev = _dot(_cat([dec_ev, encbf[:, 0:128]]), wq_ref[...]) + bqv
    z_od = _dot(_cat([dec_od, encbf[:, 128:256]]), wq_ref[...]) + bqv
    en = en_ref[0][None, :]
    iota = lax.broadcasted_iota(jnp.int32, (t, _NE), 1)
    qs = []
    dsum = jnp.zeros((), jnp.float32)
    for z in (z_ev, z_od):
        d = en - 2.0 * _dot(z, emb_ref[...])
        m = jnp.min(d, axis=1, keepdims=True)
        idx = jnp.min(jnp.where(d == m, iota, _NE), axis=1, keepdims=True)
        oh = (iota == idx).astype(jnp.float32)
        q = _dotf(oh, embt_ref[...])
        qs.append(q)
        dsum = dsum + jnp.sum((q - z) ** 2)
    q_ev, q_od = qs

    @pl.when(pl.program_id(0) == 0)
    def _():
        diff_ref[...] = jnp.zeros((1, 1), jnp.float32)

    diff_ref[...] += dsum.reshape(1, 1)

    bu = bup_ref[0][None, :]
    up_ev = _dot(_cat([x[1:1 + t], x[0:t]]), upev_ref[...]) + bu
    up_od = _dot(_cat([x[1:1 + t], x[2:2 + t]]), upod_ref[...]) + bu

    catf = _cat([up_ev, q_ev, up_od, q_od])
    cp = _vpad(catf)
    bd0 = bd0_ref[0][None, :]
    h_ev = _dot(_cat([cp[0:t, 128:256], cp[1:1 + t, 0:128],
                      cp[1:1 + t, 128:256]]), wd0_ref[...]) + bd0
    h_od = _dot(_cat([cp[1:1 + t, 0:128], cp[1:1 + t, 128:256],
                      cp[2:2 + t, 0:128]]), wd0_ref[...]) + bd0
    hf = _cat([h_ev, h_od])

    for w1r, b1r, w2r, b2r in ((s0w1_ref, s0b1_ref, s0w2_ref, s0b2_ref),
                               (s1w1_ref, s1b1_ref, s1w2_ref, s1b2_ref)):
        ap = jnp.maximum(_vpad(hf), 0.0)
        b1v = b1r[0][None, :]
        a_ev = jnp.maximum(
            _dot(_cat([ap[0:t, 128:256], ap[1:1 + t, 0:128],
                       ap[1:1 + t, 128:256]]), w1r[...]) + b1v, 0.0)
        a_od = jnp.maximum(
            _dot(_cat([ap[1:1 + t, 0:128], ap[1:1 + t, 128:256],
                       ap[2:2 + t, 0:128]]), w1r[...]) + b1v, 0.0)
        b2v = b2r[0][None, :]
        hf = hf + _cat([_dot(a_ev, w2r[...]) + b2v,
                        _dot(a_od, w2r[...]) + b2v])

    gp = _vpad(jnp.maximum(hf, 0.0))
    b1t = bt1_ref[0][None, :]
    o0 = _dot(_cat([gp[1:1 + t, 0:128], gp[0:t, 128:256]]), d1ev_ref[...]) + b1t
    o1 = _dot(_cat([gp[1:1 + t, 0:128], gp[1:1 + t, 128:256]]), d1od_ref[...]) + b1t
    o2 = _dot(_cat([gp[1:1 + t, 128:256], gp[1:1 + t, 0:128]]), d1ev_ref[...]) + b1t
    o3 = _dot(_cat([gp[1:1 + t, 128:256], gp[2:2 + t, 0:128]]), d1od_ref[...]) + b1t
    o_ref[0] = jnp.maximum(_cat([o0, o1, o2, o3]), 0.0)


def _mk_tconv2_body(post_relu):
    def body(x_ref, wev_ref, wod_ref, b_ref, o_ref):
        x = x_ref[0]
        t = o_ref.shape[1]
        bb = b_ref[0][None, :]
        ev = _dot(_cat([x[1:1 + t], x[0:t]]), wev_ref[...]) + bb
        od = _dot(_cat([x[1:1 + t], x[2:2 + t]]), wod_ref[...]) + bb
        acc = _cat([ev, od])
        if post_relu:
            acc = jnp.maximum(acc, 0.0)
        o_ref[0] = acc
    return body


def _pad1(x):
    return jnp.pad(x, ((0, 0), (1, 1), (0, 0)))


def _fold(x):
    b, t, c = x.shape
    return x.reshape(b, t // 2, 2 * c)


def _wcol_conv(w):
    k = w.shape[2]
    wt = jnp.transpose(w, (2, 1, 0))
    return jnp.concatenate([wt[j] for j in range(k)], axis=0)


def _wcol_tconv(w):
    wf = jnp.flip(w, -1)
    return jnp.concatenate([wf[:, :, j] for j in range(4)], axis=0)


def _w2tap(w):
    wev = jnp.concatenate([w[:, :, 1], w[:, :, 3]], axis=0)
    wod = jnp.concatenate([w[:, :, 2], w[:, :, 0]], axis=0)
    return wev, wod


def _b2(b):
    return b.reshape(1, -1)


def _rb(p, pre, i):
    return (_wcol_conv(p['%s_r%d_w1' % (pre, i)]), _b2(p['%s_r%d_b1' % (pre, i)]),
            p['%s_r%d_w2' % (pre, i)][:, :, 0].T, _b2(p['%s_r%d_b2' % (pre, i)]))


def kernel(input, params):
    p = params
    B = input.shape[0]
    f32 = jnp.float32
    x = jnp.transpose(input, (0, 2, 1))

    h = pl.pallas_call(
        _eb0_body, grid=(B,),
        in_specs=[_bspec(4098, 6), _wspec(12, 64), _wspec(1, 64)],
        out_specs=_bspec(4096, 64),
        out_shape=jax.ShapeDtypeStruct((B, 4096, 64), f32),
        interpret=_INTERPRET,
    )(_pad1(_fold(x)), _wcol_conv(p['eb_w0']), _b2(p['eb_b0']))

    enc_b_pre = pl.pallas_call(
        _encb_body, grid=(B,),
        in_specs=[_bspec(2050, 128), _wspec(256, 128), _wspec(1, 128),
                  _wspec(384, 128), _wspec(1, 128),
                  _wspec(384, 32), _wspec(1, 32), _wspec(32, 128), _wspec(1, 128),
                  _wspec(384, 32), _wspec(1, 32), _wspec(32, 128), _wspec(1, 128)],
        out_specs=_bspec(2048, 128),
        out_shape=jax.ShapeDtypeStruct((B, 2048, 128), f32),
        interpret=_INTERPRET,
    )(_pad1(_fold(h)), _wcol_conv(p['eb_w1']), _b2(p['eb_b1']),
      _wcol_conv(p['eb_w2']), _b2(p['eb_b2']),
      *_rb(p, 'eb', 0), *_rb(p, 'eb', 1))

    en_t = jnp.sum(p['embed_t'] * p['embed_t'], axis=0).reshape(1, -1)
    quant_t, dt_sum = pl.pallas_call(
        _enct_body, grid=(B,),
        in_specs=[_bspec(1026, 256), _wspec(512, 64), _wspec(1, 64),
                  _wspec(192, 128), _wspec(1, 128),
                  _wspec(384, 32), _wspec(1, 32), _wspec(32, 128), _wspec(1, 128),
                  _wspec(384, 32), _wspec(1, 32), _wspec(32, 128), _wspec(1, 128),
                  _wspec(128, 64), _wspec(1, 64),
                  _wspec(64, _NE), _wspec(_NE, 64), _wspec(1, _NE)],
        out_specs=[_bspec(1024, 64), pl.BlockSpec((1, 1), lambda i: (0, 0))],
        out_shape=[jax.ShapeDtypeStruct((B, 1024, 64), f32),
                   jax.ShapeDtypeStruct((1, 1), f32)],
        interpret=_INTERPRET,
    )(_pad1(_fold(enc_b_pre)), _wcol_conv(p['et_w0']), _b2(p['et_b0']),
      _wcol_conv(p['et_w1']), _b2(p['et_b1']),
      *_rb(p, 'et', 0), *_rb(p, 'et', 1),
      p['qct_w'][:, :, 0].T, _b2(p['qct_b']),
      p['embed_t'], p['embed_t'].T, en_t)

    en_b = jnp.sum(p['embed_b'] * p['embed_b'], axis=0).reshape(1, -1)
    upev, upod = _w2tap(p['up_wt'])
    d1ev, d1od = _w2tap(p['d_wt1'])
    h, db_sum = pl.pallas_call(
        _mega_body, grid=(B,),
        in_specs=[_bspec(1026, 64), _bspec(1026, 256),
                  _wspec(192, 128), _wspec(1, 128),
                  _wspec(384, 32), _wspec(1, 32), _wspec(32, 128), _wspec(1, 128),
                  _wspec(384, 32), _wspec(1, 32), _wspec(32, 128), _wspec(1, 128),
                  _wspec(512, 64), _wspec(1, 64),
                  _wspec(192, 64), _wspec(1, 64),
                  _wspec(64, _NE), _wspec(_NE, 64), _wspec(1, _NE),
                  _wspec(128, 64), _wspec(128, 64), _wspec(1, 64),
                  _wspec(384, 128), _wspec(1, 128),
                  _wspec(384, 32), _wspec(1, 32), _wspec(32, 128), _wspec(1, 128),
                  _wspec(384, 32), _wspec(1, 32), _wspec(32, 128), _wspec(1, 128),
                  _wspec(256, 64), _wspec(256, 64), _wspec(1, 64)],
        out_specs=[_bspec(1024, 256), pl.BlockSpec((1, 1), lambda i: (0, 0))],
        out_shape=[jax.ShapeDtypeStruct((B, 1024, 256), f32),
                   jax.ShapeDtypeStruct((1, 1), f32)],
        interpret=_INTERPRET,
    )(_pad1(quant_t), _pad1(_fold(enc_b_pre)),
      _wcol_conv(p['dt_w0']), _b2(p['dt_b0']),
      *_rb(p, 'dt', 0), *_rb(p, 'dt', 1),
      _wcol_tconv(p['dt_wt']), _b2(p['dt_bt']),
      p['qcb_w'][:, :, 0].T, _b2(p['qcb_b']),
      p['embed_b'], p['embed_b'].T, en_b,
      upev, upod, _b2(p['up_bt']),
      _wcol_conv(p['d_w0']), _b2(p['d_b0']),
      *_rb(p, 'd', 0), *_rb(p, 'd', 1),
      d1ev, d1od, _b2(p['d_bt1']))
    h = h.reshape(B, 4096, 64)

    d2ev, d2od = _w2tap(p['d_wt2'])
    h = pl.pallas_call(
        _mk_tconv2_body(False), grid=(B,),
        in_specs=[_bspec(4098, 64), _wspec(128, 3), _wspec(128, 3),
                  _wspec(1, 3)],
        out_specs=_bspec(4096, 6),
        out_shape=jax.ShapeDtypeStruct((B, 4096, 6), f32),
        interpret=_INTERPRET,
    )(_pad1(h), d2ev, d2od, _b2(p['d_bt2']))
    dec = h.reshape(B, 8192, 3).transpose(0, 2, 1)

    diff = (dt_sum[0, 0] / (B * 1024 * 64)
            + db_sum[0, 0] / (B * 2048 * 64)).reshape(1)
    return dec, diff


# speedup vs baseline: 1.7990x; 1.1598x over previous
import jax
import jax.numpy as jnp
from jax import lax
from jax.experimental import pallas as pl

_INTERPRET = False

_NE = 512


def _bf(a):
    return a.astype(jnp.bfloat16)


def _dot(a, b):
    return lax.dot_general(_bf(a), _bf(b), (((1,), (0,)), ((), ())),
                           preferred_element_type=jnp.float32)


def _dotf(a, b):
    return lax.dot_general(a, b, (((1,), (0,)), ((), ())),
                           preferred_element_type=jnp.float32)


def _cat(parts):
    return jnp.concatenate(parts, axis=1)


def _cat3(x, t):
    return _cat([x[0:t], x[1:1 + t], x[2:2 + t]])


def _scol(pp, t, c):
    return _cat([pp[0:t, c:2 * c], pp[1:1 + t, 0:c],
                 pp[1:1 + t, c:2 * c], pp[2:2 + t, 0:c]])


def _vpad(h):
    z = jnp.zeros((1, h.shape[1]), jnp.float32)
    return jnp.concatenate([z, h, z], axis=0)


def _res_step(h, t, w1col_ref, b1_ref, w2_ref, b2_ref):
    hp = _vpad(h)
    ha = jnp.maximum(hp, 0.0)
    a = jnp.maximum(_dot(_cat3(ha, t), w1col_ref[...]) + b1_ref[0][None, :],
                    0.0)
    return h + (_dot(a, w2_ref[...]) + b2_ref[0][None, :])


def _quant_common(z, emb_ref, embt_ref, en_ref, q_ref, diff_ref):
    d = en_ref[0][None, :] - 2.0 * _dot(z, emb_ref[...])
    m = jnp.min(d, axis=1, keepdims=True)
    iota = lax.broadcasted_iota(jnp.int32, d.shape, 1)
    idx = jnp.min(jnp.where(d == m, iota, _NE), axis=1, keepdims=True)
    oh = (iota == idx).astype(jnp.float32)
    q = _dotf(oh, embt_ref[...])
    q_ref[0] = q

    @pl.when(pl.program_id(0) == 0)
    def _():
        diff_ref[...] = jnp.zeros((1, 1), jnp.float32)

    diff_ref[...] += jnp.sum((q - z) ** 2).reshape(1, 1)


def _wspec(*shape):
    n = len(shape)
    return pl.BlockSpec(shape, lambda i, _n=n: (0,) * _n)


def _bspec(t, c):
    return pl.BlockSpec((1, t, c), lambda i: (i, 0, 0))


def _encb_body(p_ref, we0_ref, be0_ref, w0_ref, b0_ref, w1_ref, b1_ref,
               r0w1_ref, r0b1_ref, r0w2_ref, r0b2_ref,
               r1w1_ref, r1b1_ref, r1w2_ref, r1b2_ref, o_ref):
    t = o_ref.shape[1]
    xp = p_ref[0]
    be0 = be0_ref[0][None, :]
    ev = _cat([xp[0:t, 9:12], xp[1:1 + t, 0:3],
               xp[1:1 + t, 3:6], xp[1:1 + t, 6:9]])
    od = _cat([xp[1:1 + t, 3:6], xp[1:1 + t, 6:9],
               xp[1:1 + t, 9:12], xp[2:2 + t, 0:3]])
    h_ev = jnp.maximum(_dot(ev, we0_ref[...]) + be0, 0.0)
    h_od = jnp.maximum(_dot(od, we0_ref[...]) + be0, 0.0)
    pp = _vpad(_cat([h_ev, h_od]))
    xcol = _scol(pp, t, 64)
    h = jnp.maximum(_dot(xcol, w0_ref[...]) + b0_ref[0][None, :], 0.0)
    h = _dot(_cat3(_vpad(h), t), w1_ref[...]) + b1_ref[0][None, :]
    h = _res_step(h, t, r0w1_ref, r0b1_ref, r0w2_ref, r0b2_ref)
    h = _res_step(h, t, r1w1_ref, r1b1_ref, r1w2_ref, r1b2_ref)
    o_ref[0] = h


def _enct_part(pp, t, w0_ref, b0_ref, w1_ref, b1_ref,
               r0w1_ref, r0b1_ref, r0w2_ref, r0b2_ref,
               r1w1_ref, r1b1_ref, r1w2_ref, r1b2_ref,
               wq_ref, bq_ref, emb_ref, embt_ref, en_ref, diff_ref):
    ppr = jnp.maximum(pp, 0.0)
    xcol = _scol(ppr, t, 128)
    h = jnp.maximum(_dot(xcol, w0_ref[...]) + b0_ref[0][None, :], 0.0)
    h = _dot(_cat3(_vpad(h), t), w1_ref[...]) + b1_ref[0][None, :]
    h = _res_step(h, t, r0w1_ref, r0b1_ref, r0w2_ref, r0b2_ref)
    h = _res_step(h, t, r1w1_ref, r1b1_ref, r1w2_ref, r1b2_ref)
    z = _dot(jnp.maximum(h, 0.0), wq_ref[...]) + bq_ref[0][None, :]
    d = en_ref[0][None, :] - 2.0 * _dot(z, emb_ref[...])
    m = jnp.min(d, axis=1, keepdims=True)
    iota = lax.broadcasted_iota(jnp.int32, d.shape, 1)
    idx = jnp.min(jnp.where(d == m, iota, _NE), axis=1, keepdims=True)
    oh = (iota == idx).astype(jnp.float32)
    q = _dotf(oh, embt_ref[...])

    @pl.when(pl.program_id(0) == 0)
    def _():
        diff_ref[...] = jnp.zeros((1, 1), jnp.float32)

    diff_ref[...] += jnp.sum((q - z) ** 2).reshape(1, 1)
    return q


def _mega_body(p2_ref,
               tw0_ref, tb0_ref, tw1_ref, tb1_ref,
               t0w1_ref, t0b1_ref, t0w2_ref, t0b2_ref,
               t1w1_ref, t1b1_ref, t1w2_ref, t1b2_ref,
               twq_ref, tbq_ref, temb_ref, tembt_ref, ten_ref,
               w0_ref, b0_ref,
               r0w1_ref, r0b1_ref, r0w2_ref, r0b2_ref,
               r1w1_ref, r1b1_ref, r1w2_ref, r1b2_ref,
               wt_ref, bt_ref,
               wq_ref, bq_ref, emb_ref, embt_ref, en_ref,
               upev_ref, upod_ref, bup_ref,
               wd0_ref, bd0_ref,
               s0w1_ref, s0b1_ref, s0w2_ref, s0b2_ref,
               s1w1_ref, s1b1_ref, s1w2_ref, s1b2_ref,
               d1ev_ref, d1od_ref, bt1_ref,
               o_ref, difft_ref, diff_ref):
    t = o_ref.shape[1]
    qt = _enct_part(p2_ref[0], t, tw0_ref, tb0_ref, tw1_ref, tb1_ref,
                    t0w1_ref, t0b1_ref, t0w2_ref, t0b2_ref,
                    t1w1_ref, t1b1_ref, t1w2_ref, t1b2_ref,
                    twq_ref, tbq_ref, temb_ref, tembt_ref, ten_ref,
                    difft_ref)
    x = _vpad(qt)

    h = _dot(_cat3(x, t), w0_ref[...]) + b0_ref[0][None, :]
    h = _res_step(h, t, r0w1_ref, r0b1_ref, r0w2_ref, r0b2_ref)
    h = _res_step(h, t, r1w1_ref, r1b1_ref, r1w2_ref, r1b2_ref)
    hp = _vpad(jnp.maximum(h, 0.0))
    z128 = jnp.zeros((t, 128), jnp.float32)
    bb = bt_ref[0][None, :]
    dec_ev = _dot(_cat([hp[0:t], z128, hp[1:1 + t], z128]), wt_ref[...]) + bb
    dec_od = _dot(_cat([z128, hp[1:1 + t], z128, hp[2:2 + t]]), wt_ref[...]) + bb

    encbf = jnp.maximum(p2_ref[0][1:1 + t], 0.0)
    bqv = bq_ref[0][None, :]
    z_ev = _dot(_cat([dec_ev, encbf[:, 0:128]]), wq_ref[...]) + bqv
    z_od = _dot(_cat([dec_od, encbf[:, 128:256]]), wq_ref[...]) + bqv
    en = en_ref[0][None, :]
    iota = lax.broadcasted_iota(jnp.int32, (t, _NE), 1)
    qs = []
    dsum = jnp.zeros((), jnp.float32)
    for z in (z_ev, z_od):
        d = en - 2.0 * _dot(z, emb_ref[...])
        m = jnp.min(d, axis=1, keepdims=True)
        idx = jnp.min(jnp.where(d == m, iota, _NE), axis=1, keepdims=True)
        oh = (iota == idx).astype(jnp.float32)
        q = _dotf(oh, embt_ref[...])
        qs.append(q)
        dsum = dsum + jnp.sum((q - z) ** 2)
    q_ev, q_od = qs

    @pl.when(pl.program_id(0) == 0)
    def _():
        diff_ref[...] = jnp.zeros((1, 1), jnp.float32)

    diff_ref[...] += dsum.reshape(1, 1)

    bu = bup_ref[0][None, :]
    up_ev = _dot(_cat([x[1:1 + t], x[0:t]]), upev_ref[...]) + bu
    up_od = _dot(_cat([x[1:1 + t], x[2:2 + t]]), upod_ref[...]) + bu

    catf = _cat([up_ev, q_ev, up_od, q_od])
    cp = _vpad(catf)
    bd0 = bd0_ref[0][None, :]
    h_ev = _dot(_cat([cp[0:t, 128:256], cp[1:1 + t, 0:128],
                      cp[1:1 + t, 128:256]]), wd0_ref[...]) + bd0
    h_od = _dot(_cat([cp[1:1 + t, 0:128], cp[1:1 + t, 128:256],
                      cp[2:2 + t, 0:128]]), wd0_ref[...]) + bd0
    hf = _cat([h_ev, h_od])

    for w1r, b1r, w2r, b2r in ((s0w1_ref, s0b1_ref, s0w2_ref, s0b2_ref),
                               (s1w1_ref, s1b1_ref, s1w2_ref, s1b2_ref)):
        ap = jnp.maximum(_vpad(hf), 0.0)
        b1v = b1r[0][None, :]
        a_ev = jnp.maximum(
            _dot(_cat([ap[0:t, 128:256], ap[1:1 + t, 0:128],
                       ap[1:1 + t, 128:256]]), w1r[...]) + b1v, 0.0)
        a_od = jnp.maximum(
            _dot(_cat([ap[1:1 + t, 0:128], ap[1:1 + t, 128:256],
                       ap[2:2 + t, 0:128]]), w1r[...]) + b1v, 0.0)
        b2v = b2r[0][None, :]
        hf = hf + _cat([_dot(a_ev, w2r[...]) + b2v,
                        _dot(a_od, w2r[...]) + b2v])

    gp = _vpad(jnp.maximum(hf, 0.0))
    b1t = bt1_ref[0][None, :]
    o0 = _dot(_cat([gp[1:1 + t, 0:128], gp[0:t, 128:256]]), d1ev_ref[...]) + b1t
    o1 = _dot(_cat([gp[1:1 + t, 0:128], gp[1:1 + t, 128:256]]), d1od_ref[...]) + b1t
    o2 = _dot(_cat([gp[1:1 + t, 128:256], gp[1:1 + t, 0:128]]), d1ev_ref[...]) + b1t
    o3 = _dot(_cat([gp[1:1 + t, 128:256], gp[2:2 + t, 0:128]]), d1od_ref[...]) + b1t
    o_ref[0] = jnp.maximum(_cat([o0, o1, o2, o3]), 0.0)


def _mk_tconv2_body(post_relu):
    def body(x_ref, wev_ref, wod_ref, b_ref, o_ref):
        x = x_ref[0]
        t = o_ref.shape[1]
        bb = b_ref[0][None, :]
        ev = _dot(_cat([x[1:1 + t], x[0:t]]), wev_ref[...]) + bb
        od = _dot(_cat([x[1:1 + t], x[2:2 + t]]), wod_ref[...]) + bb
        acc = _cat([ev, od])
        if post_relu:
            acc = jnp.maximum(acc, 0.0)
        o_ref[0] = acc
    return body


def _pad1(x):
    return jnp.pad(x, ((0, 0), (1, 1), (0, 0)))


def _fold(x):
    b, t, c = x.shape
    return x.reshape(b, t // 2, 2 * c)


def _wcol_conv(w):
    k = w.shape[2]
    wt = jnp.transpose(w, (2, 1, 0))
    return jnp.concatenate([wt[j] for j in range(k)], axis=0)


def _wcol_tconv(w):
    wf = jnp.flip(w, -1)
    return jnp.concatenate([wf[:, :, j] for j in range(4)], axis=0)


def _w2tap(w):
    wev = jnp.concatenate([w[:, :, 1], w[:, :, 3]], axis=0)
    wod = jnp.concatenate([w[:, :, 2], w[:, :, 0]], axis=0)
    return wev, wod


def _b2(b):
    return b.reshape(1, -1)


def _rb(p, pre, i):
    return (_wcol_conv(p['%s_r%d_w1' % (pre, i)]), _b2(p['%s_r%d_b1' % (pre, i)]),
            p['%s_r%d_w2' % (pre, i)][:, :, 0].T, _b2(p['%s_r%d_b2' % (pre, i)]))


def kernel(input, params):
    p = params
    B = input.shape[0]
    f32 = jnp.float32
    x = jnp.transpose(input, (0, 2, 1))

    enc_b_pre = pl.pallas_call(
        _encb_body, grid=(B,),
        in_specs=[_bspec(2050, 12), _wspec(12, 64), _wspec(1, 64),
                  _wspec(256, 128), _wspec(1, 128),
                  _wspec(384, 128), _wspec(1, 128),
                  _wspec(384, 32), _wspec(1, 32), _wspec(32, 128), _wspec(1, 128),
                  _wspec(384, 32), _wspec(1, 32), _wspec(32, 128), _wspec(1, 128)],
        out_specs=_bspec(2048, 128),
        out_shape=jax.ShapeDtypeStruct((B, 2048, 128), f32),
        interpret=_INTERPRET,
    )(_pad1(x.reshape(B, 2048, 12)), _wcol_conv(p['eb_w0']), _b2(p['eb_b0']),
      _wcol_conv(p['eb_w1']), _b2(p['eb_b1']),
      _wcol_conv(p['eb_w2']), _b2(p['eb_b2']),
      *_rb(p, 'eb', 0), *_rb(p, 'eb', 1))

    en_t = jnp.sum(p['embed_t'] * p['embed_t'], axis=0).reshape(1, -1)
    en_b = jnp.sum(p['embed_b'] * p['embed_b'], axis=0).reshape(1, -1)
    upev, upod = _w2tap(p['up_wt'])
    d1ev, d1od = _w2tap(p['d_wt1'])
    h, dt_sum, db_sum = pl.pallas_call(
        _mega_body, grid=(B,),
        in_specs=[_bspec(1026, 256),
                  _wspec(512, 64), _wspec(1, 64),
                  _wspec(192, 128), _wspec(1, 128),
                  _wspec(384, 32), _wspec(1, 32), _wspec(32, 128), _wspec(1, 128),
                  _wspec(384, 32), _wspec(1, 32), _wspec(32, 128), _wspec(1, 128),
                  _wspec(128, 64), _wspec(1, 64),
                  _wspec(64, _NE), _wspec(_NE, 64), _wspec(1, _NE),
                  _wspec(192, 128), _wspec(1, 128),
                  _wspec(384, 32), _wspec(1, 32), _wspec(32, 128), _wspec(1, 128),
                  _wspec(384, 32), _wspec(1, 32), _wspec(32, 128), _wspec(1, 128),
                  _wspec(512, 64), _wspec(1, 64),
                  _wspec(192, 64), _wspec(1, 64),
                  _wspec(64, _NE), _wspec(_NE, 64), _wspec(1, _NE),
                  _wspec(128, 64), _wspec(128, 64), _wspec(1, 64),
                  _wspec(384, 128), _wspec(1, 128),
                  _wspec(384, 32), _wspec(1, 32), _wspec(32, 128), _wspec(1, 128),
                  _wspec(384, 32), _wspec(1, 32), _wspec(32, 128), _wspec(1, 128),
                  _wspec(256, 64), _wspec(256, 64), _wspec(1, 64)],
        out_specs=[_bspec(1024, 256),
                   pl.BlockSpec((1, 1), lambda i: (0, 0)),
                   pl.BlockSpec((1, 1), lambda i: (0, 0))],
        out_shape=[jax.ShapeDtypeStruct((B, 1024, 256), f32),
                   jax.ShapeDtypeStruct((1, 1), f32),
                   jax.ShapeDtypeStruct((1, 1), f32)],
        interpret=_INTERPRET,
    )(_pad1(_fold(enc_b_pre)),
      _wcol_conv(p['et_w0']), _b2(p['et_b0']),
      _wcol_conv(p['et_w1']), _b2(p['et_b1']),
      *_rb(p, 'et', 0), *_rb(p, 'et', 1),
      p['qct_w'][:, :, 0].T, _b2(p['qct_b']),
      p['embed_t'], p['embed_t'].T, en_t,
      _wcol_conv(p['dt_w0']), _b2(p['dt_b0']),
      *_rb(p, 'dt', 0), *_rb(p, 'dt', 1),
      _wcol_tconv(p['dt_wt']), _b2(p['dt_bt']),
      p['qcb_w'][:, :, 0].T, _b2(p['qcb_b']),
      p['embed_b'], p['embed_b'].T, en_b,
      upev, upod, _b2(p['up_bt']),
      _wcol_conv(p['d_w0']), _b2(p['d_b0']),
      *_rb(p, 'd', 0), *_rb(p, 'd', 1),
      d1ev, d1od, _b2(p['d_bt1']))
    h = h.reshape(B, 4096, 64)

    d2ev, d2od = _w2tap(p['d_wt2'])
    h = pl.pallas_call(
        _mk_tconv2_body(False), grid=(B,),
        in_specs=[_bspec(4098, 64), _wspec(128, 3), _wspec(128, 3),
                  _wspec(1, 3)],
        out_specs=_bspec(4096, 6),
        out_shape=jax.ShapeDtypeStruct((B, 4096, 6), f32),
        interpret=_INTERPRET,
    )(_pad1(h), d2ev, d2od, _b2(p['d_bt2']))
    dec = h.reshape(B, 8192, 3).transpose(0, 2, 1)

    diff = (dt_sum[0, 0] / (B * 1024 * 64)
            + db_sum[0, 0] / (B * 2048 * 64)).reshape(1)
    return dec, diff


# speedup vs baseline: 1.8023x; 1.0018x over previous
import jax
import jax.numpy as jnp
from jax import lax
from jax.experimental import pallas as pl

_NE = 512


def _bf(a):
    return a.astype(jnp.bfloat16)


def _dot(a, b):
    return lax.dot_general(_bf(a), _bf(b), (((1,), (0,)), ((), ())),
                           preferred_element_type=jnp.float32)


def _dotf(a, b):
    return lax.dot_general(a, b, (((1,), (0,)), ((), ())),
                           preferred_element_type=jnp.float32)


def _cat(parts):
    return jnp.concatenate(parts, axis=1)


def _cat3(x, t):
    return _cat([x[0:t], x[1:1 + t], x[2:2 + t]])


def _scol(pp, t, c):
    return _cat([pp[0:t, c:2 * c], pp[1:1 + t, 0:c],
                 pp[1:1 + t, c:2 * c], pp[2:2 + t, 0:c]])


def _vpad(h):
    z = jnp.zeros((1, h.shape[1]), jnp.float32)
    return jnp.concatenate([z, h, z], axis=0)


def _res_step(h, t, w1col_ref, b1_ref, w2_ref, b2_ref):
    hp = _vpad(h)
    ha = jnp.maximum(hp, 0.0)
    a = jnp.maximum(_dot(_cat3(ha, t), w1col_ref[...]) + b1_ref[0][None, :],
                    0.0)
    return h + (_dot(a, w2_ref[...]) + b2_ref[0][None, :])


def _quant_common(z, emb_ref, embt_ref, en_ref, q_ref, diff_ref):
    d = en_ref[0][None, :] - 2.0 * _dot(z, emb_ref[...])
    m = jnp.min(d, axis=1, keepdims=True)
    iota = lax.broadcasted_iota(jnp.int32, d.shape, 1)
    idx = jnp.min(jnp.where(d == m, iota, _NE), axis=1, keepdims=True)
    oh = (iota == idx).astype(jnp.float32)
    q = _dotf(oh, embt_ref[...])
    q_ref[0] = q

    @pl.when(pl.program_id(0) == 0)
    def _():
        diff_ref[...] = jnp.zeros((1, 1), jnp.float32)

    diff_ref[...] += jnp.sum((q - z) ** 2).reshape(1, 1)


def _wspec(*shape):
    n = len(shape)
    return pl.BlockSpec(shape, lambda i, _n=n: (0,) * _n)


def _bspec(t, c):
    return pl.BlockSpec((1, t, c), lambda i: (i, 0, 0))


def _encb_body(p_ref, we0_ref, be0_ref, w0_ref, b0_ref, w1_ref, b1_ref,
               r0w1_ref, r0b1_ref, r0w2_ref, r0b2_ref,
               r1w1_ref, r1b1_ref, r1w2_ref, r1b2_ref, o_ref):
    t = o_ref.shape[1]
    xp = p_ref[0]
    be0 = be0_ref[0][None, :]
    ev = _cat([xp[0:t, 9:12], xp[1:1 + t, 0:3],
               xp[1:1 + t, 3:6], xp[1:1 + t, 6:9]])
    od = _cat([xp[1:1 + t, 3:6], xp[1:1 + t, 6:9],
               xp[1:1 + t, 9:12], xp[2:2 + t, 0:3]])
    h_ev = jnp.maximum(_dot(ev, we0_ref[...]) + be0, 0.0)
    h_od = jnp.maximum(_dot(od, we0_ref[...]) + be0, 0.0)
    pp = _vpad(_cat([h_ev, h_od]))
    xcol = _scol(pp, t, 64)
    h = jnp.maximum(_dot(xcol, w0_ref[...]) + b0_ref[0][None, :], 0.0)
    h = _dot(_cat3(_vpad(h), t), w1_ref[...]) + b1_ref[0][None, :]
    h = _res_step(h, t, r0w1_ref, r0b1_ref, r0w2_ref, r0b2_ref)
    h = _res_step(h, t, r1w1_ref, r1b1_ref, r1w2_ref, r1b2_ref)
    o_ref[0] = h


def _enct_part(pp, t, w0_ref, b0_ref, w1_ref, b1_ref,
               r0w1_ref, r0b1_ref, r0w2_ref, r0b2_ref,
               r1w1_ref, r1b1_ref, r1w2_ref, r1b2_ref,
               wq_ref, bq_ref, emb_ref, embt_ref, en_ref, diff_ref):
    ppr = jnp.maximum(pp, 0.0)
    xcol = _scol(ppr, t, 128)
    h = jnp.maximum(_dot(xcol, w0_ref[...]) + b0_ref[0][None, :], 0.0)
    h = _dot(_cat3(_vpad(h), t), w1_ref[...]) + b1_ref[0][None, :]
    h = _res_step(h, t, r0w1_ref, r0b1_ref, r0w2_ref, r0b2_ref)
    h = _res_step(h, t, r1w1_ref, r1b1_ref, r1w2_ref, r1b2_ref)
    z = _dot(jnp.maximum(h, 0.0), wq_ref[...]) + bq_ref[0][None, :]
    d = en_ref[0][None, :] - 2.0 * _dot(z, emb_ref[...])
    m = jnp.min(d, axis=1, keepdims=True)
    iota = lax.broadcasted_iota(jnp.int32, d.shape, 1)
    idx = jnp.min(jnp.where(d == m, iota, _NE), axis=1, keepdims=True)
    oh = (iota == idx).astype(jnp.float32)
    q = _dotf(oh, embt_ref[...])

    @pl.when(pl.program_id(0) == 0)
    def _():
        diff_ref[...] = jnp.zeros((1, 1), jnp.float32)

    diff_ref[...] += jnp.sum((q - z) ** 2).reshape(1, 1)
    return q


def _mega_body(p2_ref,
               tw0_ref, tb0_ref, tw1_ref, tb1_ref,
               t0w1_ref, t0b1_ref, t0w2_ref, t0b2_ref,
               t1w1_ref, t1b1_ref, t1w2_ref, t1b2_ref,
               twq_ref, tbq_ref, temb_ref, tembt_ref, ten_ref,
               w0_ref, b0_ref,
               r0w1_ref, r0b1_ref, r0w2_ref, r0b2_ref,
               r1w1_ref, r1b1_ref, r1w2_ref, r1b2_ref,
               wt_ref, bt_ref,
               wq_ref, bq_ref, emb_ref, embt_ref, en_ref,
               upev_ref, upod_ref, bup_ref,
               wd0_ref, bd0_ref,
               s0w1_ref, s0b1_ref, s0w2_ref, s0b2_ref,
               s1w1_ref, s1b1_ref, s1w2_ref, s1b2_ref,
               d1ev_ref, d1od_ref, bt1_ref,
               o_ref, difft_ref, diff_ref):
    t = o_ref.shape[1]
    qt = _enct_part(p2_ref[0], t, tw0_ref, tb0_ref, tw1_ref, tb1_ref,
                    t0w1_ref, t0b1_ref, t0w2_ref, t0b2_ref,
                    t1w1_ref, t1b1_ref, t1w2_ref, t1b2_ref,
                    twq_ref, tbq_ref, temb_ref, tembt_ref, ten_ref,
                    difft_ref)
    x = _vpad(qt)

    h = _dot(_cat3(x, t), w0_ref[...]) + b0_ref[0][None, :]
    h = _res_step(h, t, r0w1_ref, r0b1_ref, r0w2_ref, r0b2_ref)
    h = _res_step(h, t, r1w1_ref, r1b1_ref, r1w2_ref, r1b2_ref)
    hp = _vpad(jnp.maximum(h, 0.0))
    z128 = jnp.zeros((t, 128), jnp.float32)
    bb = bt_ref[0][None, :]
    dec_ev = _dot(_cat([hp[0:t], z128, hp[1:1 + t], z128]), wt_ref[...]) + bb
    dec_od = _dot(_cat([z128, hp[1:1 + t], z128, hp[2:2 + t]]), wt_ref[...]) + bb

    encbf = jnp.maximum(p2_ref[0][1:1 + t], 0.0)
    bqv = bq_ref[0][None, :]
    z_ev = _dot(_cat([dec_ev, encbf[:, 0:128]]), wq_ref[...]) + bqv
    z_od = _dot(_cat([dec_od, encbf[:, 128:256]]), wq_ref[...]) + bqv
    en = en_ref[0][None, :]
    iota = lax.broadcasted_iota(jnp.int32, (t, _NE), 1)
    qs = []
    dsum = jnp.zeros((), jnp.float32)
    for z in (z_ev, z_od):
        d = en - 2.0 * _dot(z, emb_ref[...])
        m = jnp.min(d, axis=1, keepdims=True)
        idx = jnp.min(jnp.where(d == m, iota, _NE), axis=1, keepdims=True)
        oh = (iota == idx).astype(jnp.float32)
        q = _dotf(oh, embt_ref[...])
        qs.append(q)
        dsum = dsum + jnp.sum((q - z) ** 2)
    q_ev, q_od = qs

    @pl.when(pl.program_id(0) == 0)
    def _():
        diff_ref[...] = jnp.zeros((1, 1), jnp.float32)

    diff_ref[...] += dsum.reshape(1, 1)

    bu = bup_ref[0][None, :]
    up_ev = _dot(_cat([x[1:1 + t], x[0:t]]), upev_ref[...]) + bu
    up_od = _dot(_cat([x[1:1 + t], x[2:2 + t]]), upod_ref[...]) + bu

    catf = _cat([up_ev, q_ev, up_od, q_od])
    cp = _vpad(catf)
    bd0 = bd0_ref[0][None, :]
    h_ev = _dot(_cat([cp[0:t, 128:256], cp[1:1 + t, 0:128],
                      cp[1:1 + t, 128:256]]), wd0_ref[...]) + bd0
    h_od = _dot(_cat([cp[1:1 + t, 0:128], cp[1:1 + t, 128:256],
                      cp[2:2 + t, 0:128]]), wd0_ref[...]) + bd0
    hf = _cat([h_ev, h_od])

    for w1r, b1r, w2r, b2r in ((s0w1_ref, s0b1_ref, s0w2_ref, s0b2_ref),
                               (s1w1_ref, s1b1_ref, s1w2_ref, s1b2_ref)):
        ap = jnp.maximum(_vpad(hf), 0.0)
        b1v = b1r[0][None, :]
        a_ev = jnp.maximum(
            _dot(_cat([ap[0:t, 128:256], ap[1:1 + t, 0:128],
                       ap[1:1 + t, 128:256]]), w1r[...]) + b1v, 0.0)
        a_od = jnp.maximum(
            _dot(_cat([ap[1:1 + t, 0:128], ap[1:1 + t, 128:256],
                       ap[2:2 + t, 0:128]]), w1r[...]) + b1v, 0.0)
        b2v = b2r[0][None, :]
        hf = hf + _cat([_dot(a_ev, w2r[...]) + b2v,
                        _dot(a_od, w2r[...]) + b2v])

    gp = _vpad(jnp.maximum(hf, 0.0))
    b1t = bt1_ref[0][None, :]
    o0 = _dot(_cat([gp[1:1 + t, 0:128], gp[0:t, 128:256]]), d1ev_ref[...]) + b1t
    o1 = _dot(_cat([gp[1:1 + t, 0:128], gp[1:1 + t, 128:256]]), d1od_ref[...]) + b1t
    o2 = _dot(_cat([gp[1:1 + t, 128:256], gp[1:1 + t, 0:128]]), d1ev_ref[...]) + b1t
    o3 = _dot(_cat([gp[1:1 + t, 128:256], gp[2:2 + t, 0:128]]), d1od_ref[...]) + b1t
    o_ref[0] = jnp.maximum(_cat([o0, o1, o2, o3]), 0.0)


def _mk_tconv2_body(post_relu):
    def body(x_ref, wev_ref, wod_ref, b_ref, o_ref):
        x = x_ref[0]
        t = o_ref.shape[1]
        bb = b_ref[0][None, :]
        ev = _dot(_cat([x[1:1 + t], x[0:t]]), wev_ref[...]) + bb
        od = _dot(_cat([x[1:1 + t], x[2:2 + t]]), wod_ref[...]) + bb
        acc = _cat([ev, od])
        if post_relu:
            acc = jnp.maximum(acc, 0.0)
        o_ref[0] = acc
    return body


def _pad1(x):
    return jnp.pad(x, ((0, 0), (1, 1), (0, 0)))


def _fold(x):
    b, t, c = x.shape
    return x.reshape(b, t // 2, 2 * c)


def _wcol_conv(w):
    k = w.shape[2]
    wt = jnp.transpose(w, (2, 1, 0))
    return jnp.concatenate([wt[j] for j in range(k)], axis=0)


def _wcol_tconv(w):
    wf = jnp.flip(w, -1)
    return jnp.concatenate([wf[:, :, j] for j in range(4)], axis=0)


def _w2tap(w):
    wev = jnp.concatenate([w[:, :, 1], w[:, :, 3]], axis=0)
    wod = jnp.concatenate([w[:, :, 2], w[:, :, 0]], axis=0)
    return wev, wod


def _b2(b):
    return b.reshape(1, -1)


def _rb(p, pre, i):
    return (_wcol_conv(p['%s_r%d_w1' % (pre, i)]), _b2(p['%s_r%d_b1' % (pre, i)]),
            p['%s_r%d_w2' % (pre, i)][:, :, 0].T, _b2(p['%s_r%d_b2' % (pre, i)]))


def kernel(input, params):
    p = params
    B = input.shape[0]
    f32 = jnp.float32
    x = jnp.transpose(input, (0, 2, 1))

    enc_b_pre = pl.pallas_call(
        _encb_body, grid=(B,),
        in_specs=[_bspec(2050, 12), _wspec(12, 64), _wspec(1, 64),
                  _wspec(256, 128), _wspec(1, 128),
                  _wspec(384, 128), _wspec(1, 128),
                  _wspec(384, 32), _wspec(1, 32), _wspec(32, 128), _wspec(1, 128),
                  _wspec(384, 32), _wspec(1, 32), _wspec(32, 128), _wspec(1, 128)],
        out_specs=_bspec(2048, 128),
        out_shape=jax.ShapeDtypeStruct((B, 2048, 128), f32),
    )(_pad1(x.reshape(B, 2048, 12)), _wcol_conv(p['eb_w0']), _b2(p['eb_b0']),
      _wcol_conv(p['eb_w1']), _b2(p['eb_b1']),
      _wcol_conv(p['eb_w2']), _b2(p['eb_b2']),
      *_rb(p, 'eb', 0), *_rb(p, 'eb', 1))

    en_t = jnp.sum(p['embed_t'] * p['embed_t'], axis=0).reshape(1, -1)
    en_b = jnp.sum(p['embed_b'] * p['embed_b'], axis=0).reshape(1, -1)
    upev, upod = _w2tap(p['up_wt'])
    d1ev, d1od = _w2tap(p['d_wt1'])
    h, dt_sum, db_sum = pl.pallas_call(
        _mega_body, grid=(B,),
        in_specs=[_bspec(1026, 256),
                  _wspec(512, 64), _wspec(1, 64),
                  _wspec(192, 128), _wspec(1, 128),
                  _wspec(384, 32), _wspec(1, 32), _wspec(32, 128), _wspec(1, 128),
                  _wspec(384, 32), _wspec(1, 32), _wspec(32, 128), _wspec(1, 128),
                  _wspec(128, 64), _wspec(1, 64),
                  _wspec(64, _NE), _wspec(_NE, 64), _wspec(1, _NE),
                  _wspec(192, 128), _wspec(1, 128),
                  _wspec(384, 32), _wspec(1, 32), _wspec(32, 128), _wspec(1, 128),
                  _wspec(384, 32), _wspec(1, 32), _wspec(32, 128), _wspec(1, 128),
                  _wspec(512, 64), _wspec(1, 64),
                  _wspec(192, 64), _wspec(1, 64),
                  _wspec(64, _NE), _wspec(_NE, 64), _wspec(1, _NE),
                  _wspec(128, 64), _wspec(128, 64), _wspec(1, 64),
                  _wspec(384, 128), _wspec(1, 128),
                  _wspec(384, 32), _wspec(1, 32), _wspec(32, 128), _wspec(1, 128),
                  _wspec(384, 32), _wspec(1, 32), _wspec(32, 128), _wspec(1, 128),
                  _wspec(256, 64), _wspec(256, 64), _wspec(1, 64)],
        out_specs=[_bspec(1024, 256),
                   pl.BlockSpec((1, 1), lambda i: (0, 0)),
                   pl.BlockSpec((1, 1), lambda i: (0, 0))],
        out_shape=[jax.ShapeDtypeStruct((B, 1024, 256), f32),
                   jax.ShapeDtypeStruct((1, 1), f32),
                   jax.ShapeDtypeStruct((1, 1), f32)],
    )(_pad1(_fold(enc_b_pre)),
      _wcol_conv(p['et_w0']), _b2(p['et_b0']),
      _wcol_conv(p['et_w1']), _b2(p['et_b1']),
      *_rb(p, 'et', 0), *_rb(p, 'et', 1),
      p['qct_w'][:, :, 0].T, _b2(p['qct_b']),
      p['embed_t'], p['embed_t'].T, en_t,
      _wcol_conv(p['dt_w0']), _b2(p['dt_b0']),
      *_rb(p, 'dt', 0), *_rb(p, 'dt', 1),
      _wcol_tconv(p['dt_wt']), _b2(p['dt_bt']),
      p['qcb_w'][:, :, 0].T, _b2(p['qcb_b']),
      p['embed_b'], p['embed_b'].T, en_b,
      upev, upod, _b2(p['up_bt']),
      _wcol_conv(p['d_w0']), _b2(p['d_b0']),
      *_rb(p, 'd', 0), *_rb(p, 'd', 1),
      d1ev, d1od, _b2(p['d_bt1']))
    h = h.reshape(B, 4096, 64)

    d2ev, d2od = _w2tap(p['d_wt2'])
    h = pl.pallas_call(
        _mk_tconv2_body(False), grid=(B,),
        in_specs=[_bspec(4098, 64), _wspec(128, 3), _wspec(128, 3),
                  _wspec(1, 3)],
        out_specs=_bspec(4096, 6),
        out_shape=jax.ShapeDtypeStruct((B, 4096, 6), f32),
    )(_pad1(h), d2ev, d2od, _b2(p['d_bt2']))
    dec = h.reshape(B, 8192, 3).transpose(0, 2, 1)

    diff = (dt_sum[0, 0] / (B * 1024 * 64)
            + db_sum[0, 0] / (B * 2048 * 64)).reshape(1)
    return dec, diff
